# baseline (speedup 1.0000x reference)
import functools
import math

import jax
import jax.numpy as jnp
from jax import lax
from jax.experimental import pallas as pl
from jax.experimental.pallas import tpu as pltpu
from jax.experimental.pallas import tpu_sc as plsc

N_S = 100000
N_T = 100000
E = 1600000
B = 1024

NW = 32
EPW = E // NW
CH = 2000
NCH = EPW // CH
NQ = 8
QN = N_S // NQ
ACC_R = 12544
CAP = 51200
UCH = 1000
NUCH = N_S // UCH

K2 = 768
BLKE = 12800
NP = 102400
BLKN = 6400


def _mesh():
    return plsc.VectorSubcoreMesh(core_axis_name="c", subcore_axis_name="s")


def _sc_gather_bucket(tgt_hbm, src_hbm, xt_hbm, u_hbm, bs_hbm,
                      xtg_hbm, ug_hbm, bids_hbm, bsrc_hbm, bcnt_hbm,
                      xt_s, u_s, tgt_v, src_v, rows_v, fidx_v, urows_v, uidx_v,
                      lid_v, lsr_v, cnt_v, sem):
    c = lax.axis_index("c")
    s = lax.axis_index("s")
    w = s * 2 + c

    @pl.when(s == 0)
    def _stage():
        pltpu.sync_copy(xt_hbm, xt_s)
        pltpu.sync_copy(u_hbm, u_s)
    plsc.subcore_barrier()

    lane = lax.iota(jnp.int32, 16)
    wbase = w * EPW

    def chunk_body(ch, offs):
        base = pl.multiple_of(wbase + ch * CH, 8)
        pltpu.sync_copy(tgt_hbm.at[pl.ds(base, CH)], tgt_v)
        pltpu.sync_copy(src_hbm.at[pl.ds(base, CH)], src_v)

        def fidx_body(g, _):
            tv = tgt_v[pl.ds(g * 16, 16)]
            for f in range(5):
                fidx_v[f, pl.ds(g * 16, 16)] = tv + f * N_T
            return 0
        lax.fori_loop(0, CH // 16, fidx_body, 0)
        gds = [pltpu.async_copy(xt_s.at[fidx_v.at[f]], rows_v.at[f], sem)
               for f in range(5)]
        for g in gds:
            g.wait()
        wds = [pltpu.async_copy(rows_v.at[f],
                                xtg_hbm.at[pl.ds(pl.multiple_of(f * E + base, 8), CH)],
                                sem)
               for f in range(5)]
        for g in wds:
            g.wait()

        def group_body(g, offs):
            sv = src_v[pl.ds(g * 16, 16)]
            ids = base + g * 16 + lane
            new_offs = []
            for q in range(NQ):
                m = (sv >= q * QN) & (sv < (q + 1) * QN)
                mi = jnp.where(m, 1, 0).astype(jnp.int32)
                incl = plsc.cumsum(mi)
                cnt = jnp.sum(mi)
                pos = offs[q] + incl - 1
                qv = jnp.full((16,), q, jnp.int32)
                plsc.store_scatter(lid_v, [qv, pos], ids, mask=m)
                plsc.store_scatter(lsr_v, [qv, pos], sv, mask=m)
                new_offs.append(offs[q] + cnt)
            return tuple(new_offs)

        offs_c = lax.fori_loop(0, CH // 16, group_body,
                               (jnp.int32(0),) * NQ)

        for q in range(NQ):
            cnt = offs_c[q]
            cnt16 = (cnt + 15) & ~15
            pos = cnt + lane
            mpad = pos < cnt16
            qv = jnp.full((16,), q, jnp.int32)
            plsc.store_scatter(lid_v, [qv, pos], jnp.zeros((16,), jnp.int32),
                               mask=mpad)
            plsc.store_scatter(lsr_v, [qv, pos],
                               jnp.full((16,), -1, jnp.int32), mask=mpad)
            rbase = (w * NQ + q) * CAP
            dst0 = pl.multiple_of(rbase + offs[q], 8)
            pltpu.sync_copy(lid_v.at[q], bids_hbm.at[pl.ds(dst0, CH)])
            pltpu.sync_copy(lsr_v.at[q], bsrc_hbm.at[pl.ds(dst0, CH)])
        return tuple(offs[q] + ((offs_c[q] + 15) & ~15) for q in range(NQ))

    offs = lax.fori_loop(0, NCH, chunk_body, (jnp.int32(0),) * NQ)

    cvec = jnp.zeros((16,), jnp.int32)
    for q in range(NQ):
        cvec = jnp.where(lane == q, offs[q], cvec)
    cnt_v[...] = cvec
    pltpu.sync_copy(cnt_v, bcnt_hbm.at[pl.ds(pl.multiple_of(w * 16, 8), 16)])

    for k in range(4):
        t = w + k * NW
        tb = pl.multiple_of(t * UCH, 8)
        @pl.when(t < NUCH)
        def _u():
            pltpu.sync_copy(bs_hbm.at[pl.ds(tb, UCH)], uidx_v)
            pltpu.async_copy(u_s.at[uidx_v], urows_v, sem).wait()
            pltpu.sync_copy(urows_v, ug_hbm.at[pl.ds(tb, UCH)])


def _run_stage_a(tgt, src, xt_pad, u_pad, batch_s):
    f32 = jnp.float32
    i32 = jnp.int32
    out_type = (
        jax.ShapeDtypeStruct((8 * E,), f32),
        jax.ShapeDtypeStruct((N_S, 16), f32),
        jax.ShapeDtypeStruct((NW * NQ * CAP,), i32),
        jax.ShapeDtypeStruct((NW * NQ * CAP,), i32),
        jax.ShapeDtypeStruct((NW * 16,), i32),
    )
    scratch = [
        pltpu.VMEM_SHARED((8 * N_T,), f32),
        pltpu.VMEM_SHARED((B, 16), f32),
        pltpu.VMEM((CH,), i32),
        pltpu.VMEM((CH,), i32),
        pltpu.VMEM((5, CH), f32),
        pltpu.VMEM((5, CH), i32),
        pltpu.VMEM((UCH, 16), f32),
        pltpu.VMEM((UCH,), i32),
        pltpu.VMEM((NQ, CH), i32),
        pltpu.VMEM((NQ, CH), i32),
        pltpu.VMEM((16,), i32),
        pltpu.SemaphoreType.DMA,
    ]
    kfn = pl.kernel(_sc_gather_bucket, out_type=out_type, mesh=_mesh(),
                    scratch_types=scratch, name="sc_gather_bucket",
                    compiler_params=pltpu.CompilerParams(
                        needs_layout_passes=False, use_tc_tiling_on_sc=False))
    return kfn(tgt, src, xt_pad, u_pad, batch_s)


def _mlp1_body(xtT_ref, eaT_ref, w1aT_ref, b1aT_ref, w1bT_ref, b1bT_ref,
               out_ref):
    msg = jnp.concatenate([xtT_ref[...][:5], eaT_ref[...],
                           jnp.zeros((1, BLKE), jnp.float32)], axis=0)
    h = jnp.dot(w1aT_ref[...], msg, preferred_element_type=jnp.float32,
                precision=lax.Precision.HIGHEST) + b1aT_ref[...]
    h = jnp.where(h >= 0, h, 0.1 * h)
    o = jnp.dot(w1bT_ref[...], h, preferred_element_type=jnp.float32,
                precision=lax.Precision.HIGHEST) + b1bT_ref[...]
    srow = lax.broadcasted_iota(jnp.int32, (16, BLKE), 0)
    o = jnp.where(srow == 15, 1.0, o)
    out_ref[...] = o.T


def _run_stage_b(xt_gT, eaT, w1aT_p, b1aT_p, w1bT_p, b1bT_p):
    grid = (E // BLKE,)
    return pl.pallas_call(
        _mlp1_body,
        grid=grid,
        in_specs=[
            pl.BlockSpec((8, BLKE), lambda i: (0, i)),
            pl.BlockSpec((10, BLKE), lambda i: (0, i)),
            pl.BlockSpec((16, 16), lambda i: (0, 0)),
            pl.BlockSpec((16, 1), lambda i: (0, 0)),
            pl.BlockSpec((16, 16), lambda i: (0, 0)),
            pl.BlockSpec((16, 1), lambda i: (0, 0)),
        ],
        out_specs=pl.BlockSpec((BLKE, 16), lambda i: (i, 0)),
        out_shape=jax.ShapeDtypeStruct((E, 16), jnp.float32),
        compiler_params=pltpu.CompilerParams(
            dimension_semantics=("arbitrary",)),
        name="tc_edge_mlp",
    )(xt_gT, eaT, w1aT_p, b1aT_p, w1bT_p, b1bT_p)


def _sc_moments(outp_hbm, bids_hbm, bsrc_hbm, bcnt_hbm,
                s1_hbm, s2_hbm, s3_hbm, s4_hbm,
                a1, a2, a3, a4, idv, srcv, ilocv, rows, sq, cu, q4,
                cnt_v, zrow, sem):
    c = lax.axis_index("c")
    s = lax.axis_index("s")
    pltpu.sync_copy(bcnt_hbm, cnt_v)
    lane = lax.iota(jnp.int32, 16)
    stripe = ACC_R // 16

    def zinit(i, _):
        zrow[i] = jnp.zeros((16,), jnp.float32)
        return 0
    lax.fori_loop(0, stripe, zinit, 0)

    for ph in range(NQ // 2):
        q = ph * 2 + c
        qlo = q * QN

        sb0 = pl.multiple_of(s * stripe, 8)
        for acc in (a1, a2, a3, a4):
            pltpu.sync_copy(zrow, acc.at[pl.ds(sb0, stripe)])
        plsc.subcore_barrier()

        for j in range(2):
            w = s * 2 + j
            cnt = jnp.sum(jnp.where(lane == q, cnt_v[pl.ds(pl.multiple_of(w * 16, 8), 16)], 0))
            nchunks = (cnt + (K2 - 1)) // K2

            def chunk_body(k, _):
                kbase = k * K2
                rbase = pl.multiple_of((w * NQ + q) * CAP + kbase, 8)
                pltpu.sync_copy(bids_hbm.at[pl.ds(rbase, K2)], idv)
                pltpu.sync_copy(bsrc_hbm.at[pl.ds(rbase, K2)], srcv)

                def san_body(g, _):
                    pos = kbase + g * 16 + lane
                    iv = idv[pl.ds(g * 16, 16)]
                    sv = srcv[pl.ds(g * 16, 16)]
                    valid = (pos < cnt) & (sv >= 0)
                    idv[pl.ds(g * 16, 16)] = jnp.where(valid, iv, 0)
                    iloc = jnp.where(valid, sv - qlo, QN + (lane & 7))
                    ilocv[pl.ds(g * 16, 16)] = iloc
                    return 0
                lax.fori_loop(0, K2 // 16, san_body, 0)

                pltpu.async_copy(outp_hbm.at[idv], rows, sem).wait()

                def pow_body(e, _):
                    for u in range(4):
                        o = rows[e * 4 + u]
                        t2 = o * o
                        sq[e * 4 + u] = t2
                        cu[e * 4 + u] = t2 * o
                        q4[e * 4 + u] = t2 * t2
                    return 0
                lax.fori_loop(0, K2 // 4, pow_body, 0)

                d1 = pltpu.async_copy(rows, a1.at[ilocv], sem, add=True)
                d2 = pltpu.async_copy(sq, a2.at[ilocv], sem, add=True)
                d3 = pltpu.async_copy(cu, a3.at[ilocv], sem, add=True)
                d4 = pltpu.async_copy(q4, a4.at[ilocv], sem, add=True)
                d1.wait()
                d2.wait()
                d3.wait()
                d4.wait()
                return 0
            lax.fori_loop(0, nchunks, chunk_body, 0)

        plsc.subcore_barrier()
        for acc, dst in ((a1, s1_hbm), (a2, s2_hbm), (a3, s3_hbm), (a4, s4_hbm)):
            sb = pl.multiple_of(s * stripe, 8)
            pltpu.sync_copy(acc.at[pl.ds(sb, stripe)],
                            dst.at[q, pl.ds(sb, stripe)])
        plsc.subcore_barrier()


def _run_stage_c(out_p, bids, bsrc, bcnt):
    f32 = jnp.float32
    i32 = jnp.int32
    mom = jax.ShapeDtypeStruct((NQ, ACC_R, 16), f32)
    out_type = (mom, mom, mom, mom)
    scratch = [
        pltpu.VMEM_SHARED((ACC_R, 16), f32),
        pltpu.VMEM_SHARED((ACC_R, 16), f32),
        pltpu.VMEM_SHARED((ACC_R, 16), f32),
        pltpu.VMEM_SHARED((ACC_R, 16), f32),
        pltpu.VMEM((K2,), i32),
        pltpu.VMEM((K2,), i32),
        pltpu.VMEM((K2,), i32),
        pltpu.VMEM((K2, 16), f32),
        pltpu.VMEM((K2, 16), f32),
        pltpu.VMEM((K2, 16), f32),
        pltpu.VMEM((K2, 16), f32),
        pltpu.VMEM((NW * 16,), i32),
        pltpu.VMEM((ACC_R // 16, 16), f32),
        pltpu.SemaphoreType.DMA,
    ]
    kfn = pl.kernel(_sc_moments, out_type=out_type, mesh=_mesh(),
                    scratch_types=scratch, name="sc_moments",
                    compiler_params=pltpu.CompilerParams(
                        needs_layout_passes=False, use_tc_tiling_on_sc=False))
    return kfn(out_p, bids, bsrc, bcnt)


def _final_body(s1_ref, s2_ref, s3_ref, s4_ref, xsT_ref, ugT_ref,
                w2aT_ref, b2aT_ref, w2bT_ref, b2bT_ref, outT_ref):
    s1 = s1_ref[...]
    s2 = s2_ref[...]
    s3 = s3_ref[...]
    s4 = s4_ref[...]
    n = s1[15:16, :]
    cnt = jnp.maximum(n, 1.0)
    inv = 1.0 / cnt
    a = s1 * inv
    m2 = s2 * inv
    m3r = s3 * inv
    m4r = s4 * inv
    r = n * inv
    b = jnp.sqrt(1e-6 + jnp.maximum(m2 - a * a, 0.0))
    a2 = a * a
    m3 = m3r - 3.0 * a * m2 + 2.0 * a * a2 * r
    m4 = m4r - 4.0 * a * m3r + 6.0 * a2 * m2 - 4.0 * a2 * a2 + a2 * a2 * r
    b3 = b * b * b
    cmom = m3 / b3
    dmom = m4 / (b3 * b)
    sn = jnp.sqrt(n)
    cmom = jnp.where(n < 2.5, 0.0, jnp.clip(cmom, -sn, sn))
    dmom = jnp.where(n < 1.5, 0.0, jnp.clip(dmom, 0.0, n))
    feat = jnp.concatenate([
        xsT_ref[...], n, a[:15], b[:15], cmom[:15], dmom[:15],
        ugT_ref[...], jnp.zeros((7, BLKN), jnp.float32)], axis=0)
    h = jnp.dot(w2aT_ref[...], feat, preferred_element_type=jnp.float32,
                precision=lax.Precision.HIGHEST) + b2aT_ref[...]
    h = jnp.where(h >= 0, h, 0.1 * h)
    o = jnp.dot(w2bT_ref[...], h, preferred_element_type=jnp.float32,
                precision=lax.Precision.HIGHEST) + b2bT_ref[...]
    outT_ref[...] = o


def _run_stage_d(s1T, s2T, s3T, s4T, xsT, ugT, w2aT_p, b2aT_p, w2bT_p, b2bT_p):
    grid = (NP // BLKN,)
    momT_spec = pl.BlockSpec((16, BLKN), lambda i: (0, i))
    return pl.pallas_call(
        _final_body,
        grid=grid,
        in_specs=[
            momT_spec, momT_spec, momT_spec, momT_spec,
            pl.BlockSpec((10, BLKN), lambda i: (0, i)),
            pl.BlockSpec((10, BLKN), lambda i: (0, i)),
            pl.BlockSpec((16, 88), lambda i: (0, 0)),
            pl.BlockSpec((16, 1), lambda i: (0, 0)),
            pl.BlockSpec((16, 16), lambda i: (0, 0)),
            pl.BlockSpec((16, 1), lambda i: (0, 0)),
        ],
        out_specs=pl.BlockSpec((16, BLKN), lambda i: (0, i)),
        out_shape=jax.ShapeDtypeStruct((16, NP), jnp.float32),
        compiler_params=pltpu.CompilerParams(
            dimension_semantics=("arbitrary",)),
        name="tc_node_stats_mlp",
    )(s1T, s2T, s3T, s4T, xsT, ugT, w2aT_p, b2aT_p, w2bT_p, b2bT_p)


def kernel(x_s, x_t, edge_index, edge_attr, u, batch_s,
           W1a, b1a, W1b, b1b, W2a, b2a, W2b, b2b):
    src = edge_index[0]
    tgt = edge_index[1]

    xt_flat = x_t.T.reshape(-1)
    xt_pad = jnp.pad(xt_flat, (0, 3 * N_T))
    u_pad = jnp.pad(u, ((0, 0), (0, 6)))
    w1aT_p = jnp.pad(W1a.T, ((0, 1), (0, 1)))
    w1bT_p = jnp.pad(W1b.T, ((0, 1), (0, 1)))
    b1aT_p = jnp.pad(b1a, (0, 1)).reshape(16, 1)
    b1bT_p = jnp.pad(b1b, (0, 1)).reshape(16, 1)
    w2aT_p = jnp.pad(W2a.T, ((0, 6), (0, 7)))
    b2aT_p = jnp.pad(b2a, (0, 6)).reshape(16, 1)
    w2bT_p = jnp.pad(W2b.T, ((0, 6), (0, 6)))
    b2bT_p = jnp.pad(b2b, (0, 6)).reshape(16, 1)

    xt_gf, u_g, bids, bsrc, bcnt = _run_stage_a(tgt, src, xt_pad, u_pad,
                                                batch_s)
    xt_gT = xt_gf.reshape(8, E)
    eaT = edge_attr.T
    out_p = _run_stage_b(xt_gT, eaT, w1aT_p, b1aT_p, w1bT_p, b1bT_p)
    s1, s2, s3, s4 = _run_stage_c(out_p, bids, bsrc, bcnt)
    zpad = ((0, 0), (0, NP - N_S))
    s1T, s2T, s3T, s4T = (
        jnp.pad(jnp.reshape(t[:, :QN, :], (N_S, 16)).T, zpad)
        for t in (s1, s2, s3, s4))
    xsT = jnp.pad(x_s.T, zpad)
    ugT = jnp.pad(u_g.T[:10], zpad)
    outT2 = _run_stage_d(s1T, s2T, s3T, s4T, xsT, ugT,
                         w2aT_p, b2aT_p, w2bT_p, b2bT_p)
    return outT2[:10, :N_S].T


# speedup vs baseline: 1.1649x; 1.1649x over previous
import functools
import math

import jax
import jax.numpy as jnp
from jax import lax
from jax.experimental import pallas as pl
from jax.experimental.pallas import tpu as pltpu
from jax.experimental.pallas import tpu_sc as plsc

N_S = 100000
N_T = 100000
E = 1600000
B = 1024

NW = 32
EPW = E // NW
CH = 2000
NCH = EPW // CH
NQ = 8
QN = N_S // NQ
ACC_R = 12544
CAP = 51200
UCH = 1000
NUCH = N_S // UCH

K2 = 768
BLKE = 12800
NP = 102400
BLKN = 6400


def _mesh():
    return plsc.VectorSubcoreMesh(core_axis_name="c", subcore_axis_name="s")


def _sc_gather_bucket(tgt_hbm, src_hbm, xt_hbm, u_hbm, bs_hbm,
                      xtg_hbm, ug_hbm, bids_hbm, bsrc_hbm, bcnt_hbm,
                      xt_s, u_s, tgt_v, src_v, rows_v, urows_v, uidx_v,
                      lid_v, lsr_v, cnt_v, sem):
    c = lax.axis_index("c")
    s = lax.axis_index("s")
    w = s * 2 + c

    @pl.when(s == 0)
    def _stage():
        pltpu.sync_copy(xt_hbm, xt_s)
        pltpu.sync_copy(u_hbm, u_s)
    plsc.subcore_barrier()

    lane = lax.iota(jnp.int32, 16)
    wbase = w * EPW

    def chunk_body(ch, offs):
        base = pl.multiple_of(wbase + ch * CH, 8)
        pltpu.sync_copy(tgt_hbm.at[pl.ds(base, CH)], tgt_v)
        pltpu.sync_copy(src_hbm.at[pl.ds(base, CH)], src_v)
        pltpu.async_copy(xt_s.at[tgt_v], rows_v, sem).wait()
        pltpu.sync_copy(rows_v, xtg_hbm.at[pl.ds(base, CH)])

        def group_body(g, offs):
            sv = src_v[pl.ds(g * 16, 16)]
            ids = base + g * 16 + lane
            new_offs = []
            for q in range(NQ):
                m = (sv >= q * QN) & (sv < (q + 1) * QN)
                mi = jnp.where(m, 1, 0).astype(jnp.int32)
                incl = plsc.cumsum(mi)
                cnt = jnp.sum(mi)
                pos = offs[q] + incl - 1
                qv = jnp.full((16,), q, jnp.int32)
                plsc.store_scatter(lid_v, [qv, pos], ids, mask=m)
                plsc.store_scatter(lsr_v, [qv, pos], sv, mask=m)
                new_offs.append(offs[q] + cnt)
            return tuple(new_offs)

        offs_c = lax.fori_loop(0, CH // 16, group_body,
                               (jnp.int32(0),) * NQ)

        for q in range(NQ):
            cnt = offs_c[q]
            cnt16 = (cnt + 15) & ~15
            pos = cnt + lane
            mpad = pos < cnt16
            qv = jnp.full((16,), q, jnp.int32)
            plsc.store_scatter(lid_v, [qv, pos], jnp.zeros((16,), jnp.int32),
                               mask=mpad)
            plsc.store_scatter(lsr_v, [qv, pos],
                               jnp.full((16,), -1, jnp.int32), mask=mpad)
            rbase = (w * NQ + q) * CAP
            dst0 = pl.multiple_of(rbase + offs[q], 8)
            pltpu.sync_copy(lid_v.at[q], bids_hbm.at[pl.ds(dst0, CH)])
            pltpu.sync_copy(lsr_v.at[q], bsrc_hbm.at[pl.ds(dst0, CH)])
        return tuple(offs[q] + ((offs_c[q] + 15) & ~15) for q in range(NQ))

    offs = lax.fori_loop(0, NCH, chunk_body, (jnp.int32(0),) * NQ)

    cvec = jnp.zeros((16,), jnp.int32)
    for q in range(NQ):
        cvec = jnp.where(lane == q, offs[q], cvec)
    cnt_v[...] = cvec
    pltpu.sync_copy(cnt_v, bcnt_hbm.at[pl.ds(pl.multiple_of(w * 16, 8), 16)])

    for k in range(4):
        t = w + k * NW
        tb = pl.multiple_of(t * UCH, 8)
        @pl.when(t < NUCH)
        def _u():
            pltpu.sync_copy(bs_hbm.at[pl.ds(tb, UCH)], uidx_v)
            pltpu.async_copy(u_s.at[uidx_v], urows_v, sem).wait()
            pltpu.sync_copy(urows_v, ug_hbm.at[pl.ds(tb, UCH)])


def _run_stage_a(tgt, src, xt_pad, u_pad, batch_s):
    f32 = jnp.float32
    i32 = jnp.int32
    out_type = (
        jax.ShapeDtypeStruct((E, 8), f32),
        jax.ShapeDtypeStruct((N_S, 16), f32),
        jax.ShapeDtypeStruct((NW * NQ * CAP,), i32),
        jax.ShapeDtypeStruct((NW * NQ * CAP,), i32),
        jax.ShapeDtypeStruct((NW * 16,), i32),
    )
    scratch = [
        pltpu.VMEM_SHARED((N_T, 8), f32),
        pltpu.VMEM_SHARED((B, 16), f32),
        pltpu.VMEM((CH,), i32),
        pltpu.VMEM((CH,), i32),
        pltpu.VMEM((CH, 8), f32),
        pltpu.VMEM((UCH, 16), f32),
        pltpu.VMEM((UCH,), i32),
        pltpu.VMEM((NQ, CH), i32),
        pltpu.VMEM((NQ, CH), i32),
        pltpu.VMEM((16,), i32),
        pltpu.SemaphoreType.DMA,
    ]
    kfn = pl.kernel(_sc_gather_bucket, out_type=out_type, mesh=_mesh(),
                    scratch_types=scratch, name="sc_gather_bucket",
                    compiler_params=pltpu.CompilerParams(
                        needs_layout_passes=False, use_tc_tiling_on_sc=False))
    return kfn(tgt, src, xt_pad, u_pad, batch_s)


def _mlp1_body(xtT_ref, eaT_ref, w1aT_ref, b1aT_ref, w1bT_ref, b1bT_ref,
               out_ref):
    msg = jnp.concatenate([xtT_ref[...][:5], eaT_ref[...],
                           jnp.zeros((1, BLKE), jnp.float32)], axis=0)
    h = jnp.dot(w1aT_ref[...], msg, preferred_element_type=jnp.float32,
                precision=lax.Precision.HIGHEST) + b1aT_ref[...]
    h = jnp.where(h >= 0, h, 0.1 * h)
    o = jnp.dot(w1bT_ref[...], h, preferred_element_type=jnp.float32,
                precision=lax.Precision.HIGHEST) + b1bT_ref[...]
    srow = lax.broadcasted_iota(jnp.int32, (16, BLKE), 0)
    o = jnp.where(srow == 15, 1.0, o)
    out_ref[...] = o.T


def _run_stage_b(xt_gT, eaT, w1aT_p, b1aT_p, w1bT_p, b1bT_p):
    grid = (E // BLKE,)
    return pl.pallas_call(
        _mlp1_body,
        grid=grid,
        in_specs=[
            pl.BlockSpec((8, BLKE), lambda i: (0, i)),
            pl.BlockSpec((10, BLKE), lambda i: (0, i)),
            pl.BlockSpec((16, 16), lambda i: (0, 0)),
            pl.BlockSpec((16, 1), lambda i: (0, 0)),
            pl.BlockSpec((16, 16), lambda i: (0, 0)),
            pl.BlockSpec((16, 1), lambda i: (0, 0)),
        ],
        out_specs=pl.BlockSpec((BLKE, 16), lambda i: (i, 0)),
        out_shape=jax.ShapeDtypeStruct((E, 16), jnp.float32),
        compiler_params=pltpu.CompilerParams(
            dimension_semantics=("arbitrary",)),
        name="tc_edge_mlp",
    )(xt_gT, eaT, w1aT_p, b1aT_p, w1bT_p, b1bT_p)


def _sc_moments(outp_hbm, bids_hbm, bsrc_hbm, bcnt_hbm,
                s1_hbm, s2_hbm, s3_hbm, s4_hbm,
                a1, a2, a3, a4, idv, srcv, ilocv, rows, sq, cu, q4,
                cnt_v, zrow, sem):
    c = lax.axis_index("c")
    s = lax.axis_index("s")
    pltpu.sync_copy(bcnt_hbm, cnt_v)
    lane = lax.iota(jnp.int32, 16)
    stripe = ACC_R // 16

    def zinit(i, _):
        zrow[i] = jnp.zeros((16,), jnp.float32)
        return 0
    lax.fori_loop(0, stripe, zinit, 0)

    for ph in range(NQ // 2):
        q = ph * 2 + c
        qlo = q * QN

        sb0 = pl.multiple_of(s * stripe, 8)
        for acc in (a1, a2, a3, a4):
            pltpu.sync_copy(zrow, acc.at[pl.ds(sb0, stripe)])
        plsc.subcore_barrier()

        for j in range(2):
            w = s * 2 + j
            cnt = jnp.sum(jnp.where(lane == q, cnt_v[pl.ds(pl.multiple_of(w * 16, 8), 16)], 0))
            nchunks = (cnt + (K2 - 1)) // K2

            def chunk_body(k, _):
                kbase = k * K2
                rbase = pl.multiple_of((w * NQ + q) * CAP + kbase, 8)
                pltpu.sync_copy(bids_hbm.at[pl.ds(rbase, K2)], idv)
                pltpu.sync_copy(bsrc_hbm.at[pl.ds(rbase, K2)], srcv)

                def san_body(g, _):
                    pos = kbase + g * 16 + lane
                    iv = idv[pl.ds(g * 16, 16)]
                    sv = srcv[pl.ds(g * 16, 16)]
                    valid = (pos < cnt) & (sv >= 0)
                    idv[pl.ds(g * 16, 16)] = jnp.where(valid, iv, 0)
                    iloc = jnp.where(valid, sv - qlo, QN + (lane & 7))
                    ilocv[pl.ds(g * 16, 16)] = iloc
                    return 0
                lax.fori_loop(0, K2 // 16, san_body, 0)

                pltpu.async_copy(outp_hbm.at[idv], rows, sem).wait()

                def pow_body(e, _):
                    for u in range(4):
                        o = rows[e * 4 + u]
                        t2 = o * o
                        sq[e * 4 + u] = t2
                        cu[e * 4 + u] = t2 * o
                        q4[e * 4 + u] = t2 * t2
                    return 0
                lax.fori_loop(0, K2 // 4, pow_body, 0)

                d1 = pltpu.async_copy(rows, a1.at[ilocv], sem, add=True)
                d2 = pltpu.async_copy(sq, a2.at[ilocv], sem, add=True)
                d3 = pltpu.async_copy(cu, a3.at[ilocv], sem, add=True)
                d4 = pltpu.async_copy(q4, a4.at[ilocv], sem, add=True)
                d1.wait()
                d2.wait()
                d3.wait()
                d4.wait()
                return 0
            lax.fori_loop(0, nchunks, chunk_body, 0)

        plsc.subcore_barrier()
        for acc, dst in ((a1, s1_hbm), (a2, s2_hbm), (a3, s3_hbm), (a4, s4_hbm)):
            sb = pl.multiple_of(s * stripe, 8)
            pltpu.sync_copy(acc.at[pl.ds(sb, stripe)],
                            dst.at[q, pl.ds(sb, stripe)])
        plsc.subcore_barrier()


def _run_stage_c(out_p, bids, bsrc, bcnt):
    f32 = jnp.float32
    i32 = jnp.int32
    mom = jax.ShapeDtypeStruct((NQ, ACC_R, 16), f32)
    out_type = (mom, mom, mom, mom)
    scratch = [
        pltpu.VMEM_SHARED((ACC_R, 16), f32),
        pltpu.VMEM_SHARED((ACC_R, 16), f32),
        pltpu.VMEM_SHARED((ACC_R, 16), f32),
        pltpu.VMEM_SHARED((ACC_R, 16), f32),
        pltpu.VMEM((K2,), i32),
        pltpu.VMEM((K2,), i32),
        pltpu.VMEM((K2,), i32),
        pltpu.VMEM((K2, 16), f32),
        pltpu.VMEM((K2, 16), f32),
        pltpu.VMEM((K2, 16), f32),
        pltpu.VMEM((K2, 16), f32),
        pltpu.VMEM((NW * 16,), i32),
        pltpu.VMEM((ACC_R // 16, 16), f32),
        pltpu.SemaphoreType.DMA,
    ]
    kfn = pl.kernel(_sc_moments, out_type=out_type, mesh=_mesh(),
                    scratch_types=scratch, name="sc_moments",
                    compiler_params=pltpu.CompilerParams(
                        needs_layout_passes=False, use_tc_tiling_on_sc=False))
    return kfn(out_p, bids, bsrc, bcnt)


def _final_body(s1_ref, s2_ref, s3_ref, s4_ref, xsT_ref, ugT_ref,
                w2aT_ref, b2aT_ref, w2bT_ref, b2bT_ref, outT_ref):
    s1 = s1_ref[...]
    s2 = s2_ref[...]
    s3 = s3_ref[...]
    s4 = s4_ref[...]
    n = s1[15:16, :]
    cnt = jnp.maximum(n, 1.0)
    inv = 1.0 / cnt
    a = s1 * inv
    m2 = s2 * inv
    m3r = s3 * inv
    m4r = s4 * inv
    r = n * inv
    b = jnp.sqrt(1e-6 + jnp.maximum(m2 - a * a, 0.0))
    a2 = a * a
    m3 = m3r - 3.0 * a * m2 + 2.0 * a * a2 * r
    m4 = m4r - 4.0 * a * m3r + 6.0 * a2 * m2 - 4.0 * a2 * a2 + a2 * a2 * r
    b3 = b * b * b
    cmom = m3 / b3
    dmom = m4 / (b3 * b)
    sn = jnp.sqrt(n)
    cmom = jnp.where(n < 2.5, 0.0, jnp.clip(cmom, -sn, sn))
    dmom = jnp.where(n < 1.5, 0.0, jnp.clip(dmom, 0.0, n))
    feat = jnp.concatenate([
        xsT_ref[...], n, a[:15], b[:15], cmom[:15], dmom[:15],
        ugT_ref[...], jnp.zeros((7, BLKN), jnp.float32)], axis=0)
    h = jnp.dot(w2aT_ref[...], feat, preferred_element_type=jnp.float32,
                precision=lax.Precision.HIGHEST) + b2aT_ref[...]
    h = jnp.where(h >= 0, h, 0.1 * h)
    o = jnp.dot(w2bT_ref[...], h, preferred_element_type=jnp.float32,
                precision=lax.Precision.HIGHEST) + b2bT_ref[...]
    outT_ref[...] = o


def _run_stage_d(s1T, s2T, s3T, s4T, xsT, ugT, w2aT_p, b2aT_p, w2bT_p, b2bT_p):
    grid = (NP // BLKN,)
    momT_spec = pl.BlockSpec((16, BLKN), lambda i: (0, i))
    return pl.pallas_call(
        _final_body,
        grid=grid,
        in_specs=[
            momT_spec, momT_spec, momT_spec, momT_spec,
            pl.BlockSpec((10, BLKN), lambda i: (0, i)),
            pl.BlockSpec((10, BLKN), lambda i: (0, i)),
            pl.BlockSpec((16, 88), lambda i: (0, 0)),
            pl.BlockSpec((16, 1), lambda i: (0, 0)),
            pl.BlockSpec((16, 16), lambda i: (0, 0)),
            pl.BlockSpec((16, 1), lambda i: (0, 0)),
        ],
        out_specs=pl.BlockSpec((16, BLKN), lambda i: (0, i)),
        out_shape=jax.ShapeDtypeStruct((16, NP), jnp.float32),
        compiler_params=pltpu.CompilerParams(
            dimension_semantics=("arbitrary",)),
        name="tc_node_stats_mlp",
    )(s1T, s2T, s3T, s4T, xsT, ugT, w2aT_p, b2aT_p, w2bT_p, b2bT_p)


def kernel(x_s, x_t, edge_index, edge_attr, u, batch_s,
           W1a, b1a, W1b, b1b, W2a, b2a, W2b, b2b):
    src = edge_index[0]
    tgt = edge_index[1]

    xt_pad = jnp.pad(x_t, ((0, 0), (0, 3)))
    u_pad = jnp.pad(u, ((0, 0), (0, 6)))
    w1aT_p = jnp.pad(W1a.T, ((0, 1), (0, 1)))
    w1bT_p = jnp.pad(W1b.T, ((0, 1), (0, 1)))
    b1aT_p = jnp.pad(b1a, (0, 1)).reshape(16, 1)
    b1bT_p = jnp.pad(b1b, (0, 1)).reshape(16, 1)
    w2aT_p = jnp.pad(W2a.T, ((0, 6), (0, 7)))
    b2aT_p = jnp.pad(b2a, (0, 6)).reshape(16, 1)
    w2bT_p = jnp.pad(W2b.T, ((0, 6), (0, 6)))
    b2bT_p = jnp.pad(b2b, (0, 6)).reshape(16, 1)

    xt_g, u_g, bids, bsrc, bcnt = _run_stage_a(tgt, src, xt_pad, u_pad,
                                               batch_s)
    xt_gT = xt_g.T
    eaT = edge_attr.T
    out_p = _run_stage_b(xt_gT, eaT, w1aT_p, b1aT_p, w1bT_p, b1bT_p)
    s1, s2, s3, s4 = _run_stage_c(out_p, bids, bsrc, bcnt)
    zpad = ((0, 0), (0, NP - N_S))
    s1T, s2T, s3T, s4T = (
        jnp.pad(jnp.reshape(t[:, :QN, :], (N_S, 16)).T, zpad)
        for t in (s1, s2, s3, s4))
    xsT = jnp.pad(x_s.T, zpad)
    ugT = jnp.pad(u_g.T[:10], zpad)
    outT2 = _run_stage_d(s1T, s2T, s3T, s4T, xsT, ugT,
                         w2aT_p, b2aT_p, w2bT_p, b2bT_p)
    return outT2[:10, :N_S].T


# speedup vs baseline: 1.1666x; 1.0014x over previous
import functools
import math

import jax
import jax.numpy as jnp
from jax import lax
from jax.experimental import pallas as pl
from jax.experimental.pallas import tpu as pltpu
from jax.experimental.pallas import tpu_sc as plsc

N_S = 100000
N_T = 100000
E = 1600000
B = 1024

NW = 32
EPW = E // NW
CH = 2000
NCH = EPW // CH
NQ = 8
QN = N_S // NQ
ACC_R = 12544
CAP = 51200
UCH = 1000
NUCH = N_S // UCH

K2 = 768
BLKE = 12800
NP = 102400
BLKN = 6400


def _mesh():
    return plsc.VectorSubcoreMesh(core_axis_name="c", subcore_axis_name="s")


def _sc_gather_bucket(tgt_hbm, src_hbm, xt_hbm, u_hbm, bs_hbm,
                      xtg_hbm, ug_hbm, bids_hbm, bsrc_hbm, bcnt_hbm,
                      xt_s, u_s, tgt_v, src_v, rows_v, urows_v, uidx_v,
                      lid_v, lsr_v, cnt_v, sem):
    c = lax.axis_index("c")
    s = lax.axis_index("s")
    w = s * 2 + c

    @pl.when(s == 0)
    def _stage():
        pltpu.sync_copy(xt_hbm, xt_s)
        pltpu.sync_copy(u_hbm, u_s)
    plsc.subcore_barrier()

    lane = lax.iota(jnp.int32, 16)
    wbase = w * EPW

    def chunk_body(ch, offs):
        base = pl.multiple_of(wbase + ch * CH, 8)
        pltpu.sync_copy(tgt_hbm.at[pl.ds(base, CH)], tgt_v)
        pltpu.sync_copy(src_hbm.at[pl.ds(base, CH)], src_v)
        pltpu.async_copy(xt_s.at[tgt_v], rows_v, sem).wait()
        pltpu.sync_copy(rows_v, xtg_hbm.at[pl.ds(base, CH)])

        def group_body(g, offs):
            sv = src_v[pl.ds(g * 16, 16)]
            ids = base + g * 16 + lane
            new_offs = []
            for q in range(NQ):
                m = (sv >= q * QN) & (sv < (q + 1) * QN)
                mi = jnp.where(m, 1, 0).astype(jnp.int32)
                incl = plsc.cumsum(mi)
                cnt = jnp.sum(mi)
                pos = offs[q] + incl - 1
                qv = jnp.full((16,), q, jnp.int32)
                plsc.store_scatter(lid_v, [qv, pos], ids, mask=m)
                plsc.store_scatter(lsr_v, [qv, pos], sv, mask=m)
                new_offs.append(offs[q] + cnt)
            return tuple(new_offs)

        offs_c = lax.fori_loop(0, CH // 16, group_body,
                               (jnp.int32(0),) * NQ)

        for q in range(NQ):
            cnt = offs_c[q]
            cnt16 = (cnt + 15) & ~15
            pos = cnt + lane
            mpad = pos < cnt16
            qv = jnp.full((16,), q, jnp.int32)
            plsc.store_scatter(lid_v, [qv, pos], jnp.zeros((16,), jnp.int32),
                               mask=mpad)
            plsc.store_scatter(lsr_v, [qv, pos],
                               jnp.full((16,), -1, jnp.int32), mask=mpad)
            rbase = (w * NQ + q) * CAP
            dst0 = pl.multiple_of(rbase + offs[q], 8)
            pltpu.sync_copy(lid_v.at[q], bids_hbm.at[pl.ds(dst0, CH)])
            pltpu.sync_copy(lsr_v.at[q], bsrc_hbm.at[pl.ds(dst0, CH)])
        return tuple(offs[q] + ((offs_c[q] + 15) & ~15) for q in range(NQ))

    offs = lax.fori_loop(0, NCH, chunk_body, (jnp.int32(0),) * NQ)

    cvec = jnp.zeros((16,), jnp.int32)
    for q in range(NQ):
        cvec = jnp.where(lane == q, offs[q], cvec)
    cnt_v[...] = cvec
    pltpu.sync_copy(cnt_v, bcnt_hbm.at[pl.ds(pl.multiple_of(w * 16, 8), 16)])

    for k in range(4):
        t = w + k * NW
        tb = pl.multiple_of(t * UCH, 8)
        @pl.when(t < NUCH)
        def _u():
            pltpu.sync_copy(bs_hbm.at[pl.ds(tb, UCH)], uidx_v)
            pltpu.async_copy(u_s.at[uidx_v], urows_v, sem).wait()
            pltpu.sync_copy(urows_v, ug_hbm.at[pl.ds(tb, UCH)])


def _run_stage_a(tgt, src, xt_pad, u_pad, batch_s):
    f32 = jnp.float32
    i32 = jnp.int32
    out_type = (
        jax.ShapeDtypeStruct((E, 8), f32),
        jax.ShapeDtypeStruct((N_S, 16), f32),
        jax.ShapeDtypeStruct((NW * NQ * CAP,), i32),
        jax.ShapeDtypeStruct((NW * NQ * CAP,), i32),
        jax.ShapeDtypeStruct((NW * 16,), i32),
    )
    scratch = [
        pltpu.VMEM_SHARED((N_T, 8), f32),
        pltpu.VMEM_SHARED((B, 16), f32),
        pltpu.VMEM((CH,), i32),
        pltpu.VMEM((CH,), i32),
        pltpu.VMEM((CH, 8), f32),
        pltpu.VMEM((UCH, 16), f32),
        pltpu.VMEM((UCH,), i32),
        pltpu.VMEM((NQ, CH), i32),
        pltpu.VMEM((NQ, CH), i32),
        pltpu.VMEM((16,), i32),
        pltpu.SemaphoreType.DMA,
    ]
    kfn = pl.kernel(_sc_gather_bucket, out_type=out_type, mesh=_mesh(),
                    scratch_types=scratch, name="sc_gather_bucket",
                    compiler_params=pltpu.CompilerParams(
                        needs_layout_passes=False, use_tc_tiling_on_sc=False))
    return kfn(tgt, src, xt_pad, u_pad, batch_s)


def _mlp1_body(xtT_ref, eaT_ref, w1aT_ref, b1aT_ref, w1bT_ref, b1bT_ref,
               out_ref):
    msg = jnp.concatenate([xtT_ref[...][:5], eaT_ref[...],
                           jnp.zeros((1, BLKE), jnp.float32)], axis=0)
    h = jnp.dot(w1aT_ref[...], msg, preferred_element_type=jnp.float32,
                precision=lax.Precision.HIGHEST) + b1aT_ref[...]
    h = jnp.where(h >= 0, h, 0.1 * h)
    o = jnp.dot(w1bT_ref[...], h, preferred_element_type=jnp.float32,
                precision=lax.Precision.HIGHEST) + b1bT_ref[...]
    srow = lax.broadcasted_iota(jnp.int32, (16, BLKE), 0)
    o = jnp.where(srow == 15, 1.0, o)
    out_ref[...] = o.T


def _run_stage_b(xt_gT, eaT, w1aT_p, b1aT_p, w1bT_p, b1bT_p):
    grid = (E // BLKE,)
    return pl.pallas_call(
        _mlp1_body,
        grid=grid,
        in_specs=[
            pl.BlockSpec((8, BLKE), lambda i: (0, i)),
            pl.BlockSpec((10, BLKE), lambda i: (0, i)),
            pl.BlockSpec((16, 16), lambda i: (0, 0)),
            pl.BlockSpec((16, 1), lambda i: (0, 0)),
            pl.BlockSpec((16, 16), lambda i: (0, 0)),
            pl.BlockSpec((16, 1), lambda i: (0, 0)),
        ],
        out_specs=pl.BlockSpec((BLKE, 16), lambda i: (i, 0)),
        out_shape=jax.ShapeDtypeStruct((E, 16), jnp.float32),
        compiler_params=pltpu.CompilerParams(
            dimension_semantics=("arbitrary",)),
        name="tc_edge_mlp",
    )(xt_gT, eaT, w1aT_p, b1aT_p, w1bT_p, b1bT_p)


def _sc_moments(outp_hbm, bids_hbm, bsrc_hbm, bcnt_hbm,
                s1_hbm, s2_hbm, s3_hbm, s4_hbm,
                a1, a2, a3, a4, idv, srcv, ilocv, rows, sq, cu, q4,
                cnt_v, zrow, sem):
    c = lax.axis_index("c")
    s = lax.axis_index("s")
    pltpu.sync_copy(bcnt_hbm, cnt_v)
    lane = lax.iota(jnp.int32, 16)
    stripe = ACC_R // 16

    def zinit(i, _):
        zrow[i] = jnp.zeros((16,), jnp.float32)
        return 0
    lax.fori_loop(0, stripe, zinit, 0)

    for ph in range(NQ // 2):
        q = ph * 2 + c
        qlo = q * QN

        sb0 = pl.multiple_of(s * stripe, 8)
        for acc in (a1, a2, a3, a4):
            pltpu.sync_copy(zrow, acc.at[pl.ds(sb0, stripe)])
        plsc.subcore_barrier()

        for j in range(2):
            w = s * 2 + j
            cnt = jnp.sum(jnp.where(lane == q, cnt_v[pl.ds(pl.multiple_of(w * 16, 8), 16)], 0))
            nchunks = (cnt + (K2 - 1)) // K2

            def chunk_body(k, _):
                kbase = k * K2
                rbase = pl.multiple_of((w * NQ + q) * CAP + kbase, 8)
                pltpu.sync_copy(bids_hbm.at[pl.ds(rbase, K2)], idv)
                pltpu.sync_copy(bsrc_hbm.at[pl.ds(rbase, K2)], srcv)

                @plsc.parallel_loop(0, K2, step=16, unroll=2)
                def _san(g0):
                    pos = kbase + g0 + lane
                    iv = idv[pl.ds(g0, 16)]
                    sv = srcv[pl.ds(g0, 16)]
                    valid = (pos < cnt) & (sv >= 0)
                    idv[pl.ds(g0, 16)] = jnp.where(valid, iv, 0)
                    iloc = jnp.where(valid, sv - qlo, QN + (lane & 7))
                    ilocv[pl.ds(g0, 16)] = iloc

                pltpu.async_copy(outp_hbm.at[idv], rows, sem).wait()

                @plsc.parallel_loop(0, K2, step=4, unroll=4)
                def _pow(e):
                    for u in range(4):
                        o = rows[e + u]
                        t2 = o * o
                        sq[e + u] = t2
                        cu[e + u] = t2 * o
                        q4[e + u] = t2 * t2

                d1 = pltpu.async_copy(rows, a1.at[ilocv], sem, add=True)
                d2 = pltpu.async_copy(sq, a2.at[ilocv], sem, add=True)
                d3 = pltpu.async_copy(cu, a3.at[ilocv], sem, add=True)
                d4 = pltpu.async_copy(q4, a4.at[ilocv], sem, add=True)
                d1.wait()
                d2.wait()
                d3.wait()
                d4.wait()
                return 0
            lax.fori_loop(0, nchunks, chunk_body, 0)

        plsc.subcore_barrier()
        for acc, dst in ((a1, s1_hbm), (a2, s2_hbm), (a3, s3_hbm), (a4, s4_hbm)):
            sb = pl.multiple_of(s * stripe, 8)
            pltpu.sync_copy(acc.at[pl.ds(sb, stripe)],
                            dst.at[q, pl.ds(sb, stripe)])
        plsc.subcore_barrier()


def _run_stage_c(out_p, bids, bsrc, bcnt):
    f32 = jnp.float32
    i32 = jnp.int32
    mom = jax.ShapeDtypeStruct((NQ, ACC_R, 16), f32)
    out_type = (mom, mom, mom, mom)
    scratch = [
        pltpu.VMEM_SHARED((ACC_R, 16), f32),
        pltpu.VMEM_SHARED((ACC_R, 16), f32),
        pltpu.VMEM_SHARED((ACC_R, 16), f32),
        pltpu.VMEM_SHARED((ACC_R, 16), f32),
        pltpu.VMEM((K2,), i32),
        pltpu.VMEM((K2,), i32),
        pltpu.VMEM((K2,), i32),
        pltpu.VMEM((K2, 16), f32),
        pltpu.VMEM((K2, 16), f32),
        pltpu.VMEM((K2, 16), f32),
        pltpu.VMEM((K2, 16), f32),
        pltpu.VMEM((NW * 16,), i32),
        pltpu.VMEM((ACC_R // 16, 16), f32),
        pltpu.SemaphoreType.DMA,
    ]
    kfn = pl.kernel(_sc_moments, out_type=out_type, mesh=_mesh(),
                    scratch_types=scratch, name="sc_moments",
                    compiler_params=pltpu.CompilerParams(
                        needs_layout_passes=False, use_tc_tiling_on_sc=False))
    return kfn(out_p, bids, bsrc, bcnt)


def _final_body(s1_ref, s2_ref, s3_ref, s4_ref, xsT_ref, ugT_ref,
                w2aT_ref, b2aT_ref, w2bT_ref, b2bT_ref, outT_ref):
    s1 = s1_ref[...]
    s2 = s2_ref[...]
    s3 = s3_ref[...]
    s4 = s4_ref[...]
    n = s1[15:16, :]
    cnt = jnp.maximum(n, 1.0)
    inv = 1.0 / cnt
    a = s1 * inv
    m2 = s2 * inv
    m3r = s3 * inv
    m4r = s4 * inv
    r = n * inv
    b = jnp.sqrt(1e-6 + jnp.maximum(m2 - a * a, 0.0))
    a2 = a * a
    m3 = m3r - 3.0 * a * m2 + 2.0 * a * a2 * r
    m4 = m4r - 4.0 * a * m3r + 6.0 * a2 * m2 - 4.0 * a2 * a2 + a2 * a2 * r
    b3 = b * b * b
    cmom = m3 / b3
    dmom = m4 / (b3 * b)
    sn = jnp.sqrt(n)
    cmom = jnp.where(n < 2.5, 0.0, jnp.clip(cmom, -sn, sn))
    dmom = jnp.where(n < 1.5, 0.0, jnp.clip(dmom, 0.0, n))
    feat = jnp.concatenate([
        xsT_ref[...], n, a[:15], b[:15], cmom[:15], dmom[:15],
        ugT_ref[...], jnp.zeros((7, BLKN), jnp.float32)], axis=0)
    h = jnp.dot(w2aT_ref[...], feat, preferred_element_type=jnp.float32,
                precision=lax.Precision.HIGHEST) + b2aT_ref[...]
    h = jnp.where(h >= 0, h, 0.1 * h)
    o = jnp.dot(w2bT_ref[...], h, preferred_element_type=jnp.float32,
                precision=lax.Precision.HIGHEST) + b2bT_ref[...]
    outT_ref[...] = o


def _run_stage_d(s1T, s2T, s3T, s4T, xsT, ugT, w2aT_p, b2aT_p, w2bT_p, b2bT_p):
    grid = (NP // BLKN,)
    momT_spec = pl.BlockSpec((16, BLKN), lambda i: (0, i))
    return pl.pallas_call(
        _final_body,
        grid=grid,
        in_specs=[
            momT_spec, momT_spec, momT_spec, momT_spec,
            pl.BlockSpec((10, BLKN), lambda i: (0, i)),
            pl.BlockSpec((10, BLKN), lambda i: (0, i)),
            pl.BlockSpec((16, 88), lambda i: (0, 0)),
            pl.BlockSpec((16, 1), lambda i: (0, 0)),
            pl.BlockSpec((16, 16), lambda i: (0, 0)),
            pl.BlockSpec((16, 1), lambda i: (0, 0)),
        ],
        out_specs=pl.BlockSpec((16, BLKN), lambda i: (0, i)),
        out_shape=jax.ShapeDtypeStruct((16, NP), jnp.float32),
        compiler_params=pltpu.CompilerParams(
            dimension_semantics=("arbitrary",)),
        name="tc_node_stats_mlp",
    )(s1T, s2T, s3T, s4T, xsT, ugT, w2aT_p, b2aT_p, w2bT_p, b2bT_p)


def kernel(x_s, x_t, edge_index, edge_attr, u, batch_s,
           W1a, b1a, W1b, b1b, W2a, b2a, W2b, b2b):
    src = edge_index[0]
    tgt = edge_index[1]

    xt_pad = jnp.pad(x_t, ((0, 0), (0, 3)))
    u_pad = jnp.pad(u, ((0, 0), (0, 6)))
    w1aT_p = jnp.pad(W1a.T, ((0, 1), (0, 1)))
    w1bT_p = jnp.pad(W1b.T, ((0, 1), (0, 1)))
    b1aT_p = jnp.pad(b1a, (0, 1)).reshape(16, 1)
    b1bT_p = jnp.pad(b1b, (0, 1)).reshape(16, 1)
    w2aT_p = jnp.pad(W2a.T, ((0, 6), (0, 7)))
    b2aT_p = jnp.pad(b2a, (0, 6)).reshape(16, 1)
    w2bT_p = jnp.pad(W2b.T, ((0, 6), (0, 6)))
    b2bT_p = jnp.pad(b2b, (0, 6)).reshape(16, 1)

    xt_g, u_g, bids, bsrc, bcnt = _run_stage_a(tgt, src, xt_pad, u_pad,
                                               batch_s)
    xt_gT = xt_g.T
    eaT = edge_attr.T
    out_p = _run_stage_b(xt_gT, eaT, w1aT_p, b1aT_p, w1bT_p, b1bT_p)
    s1, s2, s3, s4 = _run_stage_c(out_p, bids, bsrc, bcnt)
    zpad = ((0, 0), (0, NP - N_S))
    s1T, s2T, s3T, s4T = (
        jnp.pad(jnp.reshape(t[:, :QN, :], (N_S, 16)).T, zpad)
        for t in (s1, s2, s3, s4))
    xsT = jnp.pad(x_s.T, zpad)
    ugT = jnp.pad(u_g.T[:10], zpad)
    outT2 = _run_stage_d(s1T, s2T, s3T, s4T, xsT, ugT,
                         w2aT_p, b2aT_p, w2bT_p, b2bT_p)
    return outT2[:10, :N_S].T


# speedup vs baseline: 1.1668x; 1.0002x over previous
import jax
import jax.numpy as jnp
from jax import lax
from jax.experimental import pallas as pl
from jax.experimental.pallas import tpu as pltpu
from jax.experimental.pallas import tpu_sc as plsc

N_S = 100000
N_T = 100000
E = 1600000
B = 1024

NW = 32
EPW = E // NW
CH = 2000
NCH = EPW // CH
NQ = 8
QN = N_S // NQ
ACC_R = 12544
CAP = 51200
UCH = 1000
NUCH = N_S // UCH

K2 = 768
BLKE = 12800
NP = 102400
BLKN = 6400


def _mesh():
    return plsc.VectorSubcoreMesh(core_axis_name="c", subcore_axis_name="s")


def _sc_gather_bucket(tgt_hbm, src_hbm, xt_hbm, u_hbm, bs_hbm,
                      xtg_hbm, ug_hbm, bids_hbm, bsrc_hbm, bcnt_hbm,
                      xt_s, u_s, tgt_v, src_v, rows_v, urows_v, uidx_v,
                      lid_v, lsr_v, cnt_v, sem):
    c = lax.axis_index("c")
    s = lax.axis_index("s")
    w = s * 2 + c

    @pl.when(s == 0)
    def _stage():
        pltpu.sync_copy(xt_hbm, xt_s)
        pltpu.sync_copy(u_hbm, u_s)
    plsc.subcore_barrier()

    lane = lax.iota(jnp.int32, 16)
    wbase = w * EPW

    def chunk_body(ch, offs):
        base = pl.multiple_of(wbase + ch * CH, 8)
        pltpu.sync_copy(tgt_hbm.at[pl.ds(base, CH)], tgt_v)
        pltpu.sync_copy(src_hbm.at[pl.ds(base, CH)], src_v)
        pltpu.async_copy(xt_s.at[tgt_v], rows_v, sem).wait()
        pltpu.sync_copy(rows_v, xtg_hbm.at[pl.ds(base, CH)])

        def group_body(g, offs):
            sv = src_v[pl.ds(g * 16, 16)]
            ids = base + g * 16 + lane
            new_offs = []
            for q in range(NQ):
                m = (sv >= q * QN) & (sv < (q + 1) * QN)
                mi = jnp.where(m, 1, 0).astype(jnp.int32)
                incl = plsc.cumsum(mi)
                cnt = jnp.sum(mi)
                pos = offs[q] + incl - 1
                qv = jnp.full((16,), q, jnp.int32)
                plsc.store_scatter(lid_v, [qv, pos], ids, mask=m)
                plsc.store_scatter(lsr_v, [qv, pos], sv, mask=m)
                new_offs.append(offs[q] + cnt)
            return tuple(new_offs)

        offs_c = lax.fori_loop(0, CH // 16, group_body,
                               (jnp.int32(0),) * NQ)

        for q in range(NQ):
            cnt = offs_c[q]
            cnt16 = (cnt + 15) & ~15
            pos = cnt + lane
            mpad = pos < cnt16
            qv = jnp.full((16,), q, jnp.int32)
            plsc.store_scatter(lid_v, [qv, pos], jnp.zeros((16,), jnp.int32),
                               mask=mpad)
            plsc.store_scatter(lsr_v, [qv, pos],
                               jnp.full((16,), -1, jnp.int32), mask=mpad)
            rbase = (w * NQ + q) * CAP
            dst0 = pl.multiple_of(rbase + offs[q], 8)
            pltpu.sync_copy(lid_v.at[q], bids_hbm.at[pl.ds(dst0, CH)])
            pltpu.sync_copy(lsr_v.at[q], bsrc_hbm.at[pl.ds(dst0, CH)])
        return tuple(offs[q] + ((offs_c[q] + 15) & ~15) for q in range(NQ))

    offs = lax.fori_loop(0, NCH, chunk_body, (jnp.int32(0),) * NQ)

    cvec = jnp.zeros((16,), jnp.int32)
    for q in range(NQ):
        cvec = jnp.where(lane == q, offs[q], cvec)
    cnt_v[...] = cvec
    pltpu.sync_copy(cnt_v, bcnt_hbm.at[pl.ds(pl.multiple_of(w * 16, 8), 16)])

    for k in range(4):
        t = w + k * NW
        tb = pl.multiple_of(t * UCH, 8)
        @pl.when(t < NUCH)
        def _u():
            pltpu.sync_copy(bs_hbm.at[pl.ds(tb, UCH)], uidx_v)
            pltpu.async_copy(u_s.at[uidx_v], urows_v, sem).wait()
            pltpu.sync_copy(urows_v, ug_hbm.at[pl.ds(tb, UCH)])


def _run_stage_a(tgt, src, xt_pad, u_pad, batch_s):
    f32 = jnp.float32
    i32 = jnp.int32
    out_type = (
        jax.ShapeDtypeStruct((E, 8), f32),
        jax.ShapeDtypeStruct((N_S, 16), f32),
        jax.ShapeDtypeStruct((NW * NQ * CAP,), i32),
        jax.ShapeDtypeStruct((NW * NQ * CAP,), i32),
        jax.ShapeDtypeStruct((NW * 16,), i32),
    )
    scratch = [
        pltpu.VMEM_SHARED((N_T, 8), f32),
        pltpu.VMEM_SHARED((B, 16), f32),
        pltpu.VMEM((CH,), i32),
        pltpu.VMEM((CH,), i32),
        pltpu.VMEM((CH, 8), f32),
        pltpu.VMEM((UCH, 16), f32),
        pltpu.VMEM((UCH,), i32),
        pltpu.VMEM((NQ, CH), i32),
        pltpu.VMEM((NQ, CH), i32),
        pltpu.VMEM((16,), i32),
        pltpu.SemaphoreType.DMA,
    ]
    kfn = pl.kernel(_sc_gather_bucket, out_type=out_type, mesh=_mesh(),
                    scratch_types=scratch, name="sc_gather_bucket",
                    compiler_params=pltpu.CompilerParams(
                        needs_layout_passes=False, use_tc_tiling_on_sc=False))
    return kfn(tgt, src, xt_pad, u_pad, batch_s)


def _mlp1_body(xtT_ref, eaT_ref, w1aT_ref, b1aT_ref, w1bT_ref, b1bT_ref,
               out_ref):
    msg = jnp.concatenate([xtT_ref[...][:5], eaT_ref[...],
                           jnp.zeros((1, BLKE), jnp.float32)], axis=0)
    h = jnp.dot(w1aT_ref[...], msg, preferred_element_type=jnp.float32,
                precision=lax.Precision.HIGHEST) + b1aT_ref[...]
    h = jnp.where(h >= 0, h, 0.1 * h)
    o = jnp.dot(w1bT_ref[...], h, preferred_element_type=jnp.float32,
                precision=lax.Precision.HIGHEST) + b1bT_ref[...]
    srow = lax.broadcasted_iota(jnp.int32, (16, BLKE), 0)
    o = jnp.where(srow == 15, 1.0, o)
    out_ref[...] = o.T


def _run_stage_b(xt_gT, eaT, w1aT_p, b1aT_p, w1bT_p, b1bT_p):
    grid = (E // BLKE,)
    return pl.pallas_call(
        _mlp1_body,
        grid=grid,
        in_specs=[
            pl.BlockSpec((8, BLKE), lambda i: (0, i)),
            pl.BlockSpec((10, BLKE), lambda i: (0, i)),
            pl.BlockSpec((16, 16), lambda i: (0, 0)),
            pl.BlockSpec((16, 1), lambda i: (0, 0)),
            pl.BlockSpec((16, 16), lambda i: (0, 0)),
            pl.BlockSpec((16, 1), lambda i: (0, 0)),
        ],
        out_specs=pl.BlockSpec((BLKE, 16), lambda i: (i, 0)),
        out_shape=jax.ShapeDtypeStruct((E, 16), jnp.float32),
        compiler_params=pltpu.CompilerParams(
            dimension_semantics=("arbitrary",)),
        name="tc_edge_mlp",
    )(xt_gT, eaT, w1aT_p, b1aT_p, w1bT_p, b1bT_p)


def _sc_moments(outp_hbm, bids_hbm, bsrc_hbm, bcnt_hbm,
                s1_hbm, s2_hbm, s3_hbm, s4_hbm,
                a1, a2, a3, a4, idv, srcv, ilocv, rows, sq, cu, q4,
                cnt_v, zrow, sem):
    c = lax.axis_index("c")
    s = lax.axis_index("s")
    pltpu.sync_copy(bcnt_hbm, cnt_v)
    lane = lax.iota(jnp.int32, 16)
    stripe = ACC_R // 16

    def zinit(i, _):
        zrow[i] = jnp.zeros((16,), jnp.float32)
        return 0
    lax.fori_loop(0, stripe, zinit, 0)

    for ph in range(NQ // 2):
        q = ph * 2 + c
        qlo = q * QN

        sb0 = pl.multiple_of(s * stripe, 8)
        for acc in (a1, a2, a3, a4):
            pltpu.sync_copy(zrow, acc.at[pl.ds(sb0, stripe)])
        plsc.subcore_barrier()

        for j in range(2):
            w = s * 2 + j
            cnt = jnp.sum(jnp.where(lane == q, cnt_v[pl.ds(pl.multiple_of(w * 16, 8), 16)], 0))
            nchunks = (cnt + (K2 - 1)) // K2

            def chunk_body(k, _):
                kbase = k * K2
                rbase = pl.multiple_of((w * NQ + q) * CAP + kbase, 8)
                pltpu.sync_copy(bids_hbm.at[pl.ds(rbase, K2)], idv)
                pltpu.sync_copy(bsrc_hbm.at[pl.ds(rbase, K2)], srcv)

                @plsc.parallel_loop(0, K2, step=16, unroll=2)
                def _san(g0):
                    pos = kbase + g0 + lane
                    iv = idv[pl.ds(g0, 16)]
                    sv = srcv[pl.ds(g0, 16)]
                    valid = (pos < cnt) & (sv >= 0)
                    idv[pl.ds(g0, 16)] = jnp.where(valid, iv, 0)
                    iloc = jnp.where(valid, sv - qlo, QN + (lane & 7))
                    ilocv[pl.ds(g0, 16)] = iloc

                pltpu.async_copy(outp_hbm.at[idv], rows, sem).wait()

                @plsc.parallel_loop(0, K2, step=4, unroll=4)
                def _pow(e):
                    for u in range(4):
                        o = rows[e + u]
                        t2 = o * o
                        sq[e + u] = t2
                        cu[e + u] = t2 * o
                        q4[e + u] = t2 * t2

                d1 = pltpu.async_copy(rows, a1.at[ilocv], sem, add=True)
                d2 = pltpu.async_copy(sq, a2.at[ilocv], sem, add=True)
                d3 = pltpu.async_copy(cu, a3.at[ilocv], sem, add=True)
                d4 = pltpu.async_copy(q4, a4.at[ilocv], sem, add=True)
                d1.wait()
                d2.wait()
                d3.wait()
                d4.wait()
                return 0
            lax.fori_loop(0, nchunks, chunk_body, 0)

        plsc.subcore_barrier()
        for acc, dst in ((a1, s1_hbm), (a2, s2_hbm), (a3, s3_hbm), (a4, s4_hbm)):
            sb = pl.multiple_of(s * stripe, 8)
            pltpu.sync_copy(acc.at[pl.ds(sb, stripe)],
                            dst.at[q, pl.ds(sb, stripe)])
        plsc.subcore_barrier()


def _run_stage_c(out_p, bids, bsrc, bcnt):
    f32 = jnp.float32
    i32 = jnp.int32
    mom = jax.ShapeDtypeStruct((NQ, ACC_R, 16), f32)
    out_type = (mom, mom, mom, mom)
    scratch = [
        pltpu.VMEM_SHARED((ACC_R, 16), f32),
        pltpu.VMEM_SHARED((ACC_R, 16), f32),
        pltpu.VMEM_SHARED((ACC_R, 16), f32),
        pltpu.VMEM_SHARED((ACC_R, 16), f32),
        pltpu.VMEM((K2,), i32),
        pltpu.VMEM((K2,), i32),
        pltpu.VMEM((K2,), i32),
        pltpu.VMEM((K2, 16), f32),
        pltpu.VMEM((K2, 16), f32),
        pltpu.VMEM((K2, 16), f32),
        pltpu.VMEM((K2, 16), f32),
        pltpu.VMEM((NW * 16,), i32),
        pltpu.VMEM((ACC_R // 16, 16), f32),
        pltpu.SemaphoreType.DMA,
    ]
    kfn = pl.kernel(_sc_moments, out_type=out_type, mesh=_mesh(),
                    scratch_types=scratch, name="sc_moments",
                    compiler_params=pltpu.CompilerParams(
                        needs_layout_passes=False, use_tc_tiling_on_sc=False))
    return kfn(out_p, bids, bsrc, bcnt)


def _final_body(s1_ref, s2_ref, s3_ref, s4_ref, xsT_ref, ugT_ref,
                w2aT_ref, b2aT_ref, w2bT_ref, b2bT_ref, outT_ref):
    s1 = s1_ref[...]
    s2 = s2_ref[...]
    s3 = s3_ref[...]
    s4 = s4_ref[...]
    n = s1[15:16, :]
    cnt = jnp.maximum(n, 1.0)
    inv = 1.0 / cnt
    a = s1 * inv
    m2 = s2 * inv
    m3r = s3 * inv
    m4r = s4 * inv
    r = n * inv
    b = jnp.sqrt(1e-6 + jnp.maximum(m2 - a * a, 0.0))
    a2 = a * a
    m3 = m3r - 3.0 * a * m2 + 2.0 * a * a2 * r
    m4 = m4r - 4.0 * a * m3r + 6.0 * a2 * m2 - 4.0 * a2 * a2 + a2 * a2 * r
    b3 = b * b * b
    cmom = m3 / b3
    dmom = m4 / (b3 * b)
    sn = jnp.sqrt(n)
    cmom = jnp.where(n < 2.5, 0.0, jnp.clip(cmom, -sn, sn))
    dmom = jnp.where(n < 1.5, 0.0, jnp.clip(dmom, 0.0, n))
    feat = jnp.concatenate([
        xsT_ref[...], n, a[:15], b[:15], cmom[:15], dmom[:15],
        ugT_ref[...], jnp.zeros((7, BLKN), jnp.float32)], axis=0)
    h = jnp.dot(w2aT_ref[...], feat, preferred_element_type=jnp.float32,
                precision=lax.Precision.HIGHEST) + b2aT_ref[...]
    h = jnp.where(h >= 0, h, 0.1 * h)
    o = jnp.dot(w2bT_ref[...], h, preferred_element_type=jnp.float32,
                precision=lax.Precision.HIGHEST) + b2bT_ref[...]
    outT_ref[...] = o


def _run_stage_d(s1T, s2T, s3T, s4T, xsT, ugT, w2aT_p, b2aT_p, w2bT_p, b2bT_p):
    grid = (NP // BLKN,)
    momT_spec = pl.BlockSpec((16, BLKN), lambda i: (0, i))
    return pl.pallas_call(
        _final_body,
        grid=grid,
        in_specs=[
            momT_spec, momT_spec, momT_spec, momT_spec,
            pl.BlockSpec((10, BLKN), lambda i: (0, i)),
            pl.BlockSpec((10, BLKN), lambda i: (0, i)),
            pl.BlockSpec((16, 88), lambda i: (0, 0)),
            pl.BlockSpec((16, 1), lambda i: (0, 0)),
            pl.BlockSpec((16, 16), lambda i: (0, 0)),
            pl.BlockSpec((16, 1), lambda i: (0, 0)),
        ],
        out_specs=pl.BlockSpec((16, BLKN), lambda i: (0, i)),
        out_shape=jax.ShapeDtypeStruct((16, NP), jnp.float32),
        compiler_params=pltpu.CompilerParams(
            dimension_semantics=("arbitrary",)),
        name="tc_node_stats_mlp",
    )(s1T, s2T, s3T, s4T, xsT, ugT, w2aT_p, b2aT_p, w2bT_p, b2bT_p)


def kernel(x_s, x_t, edge_index, edge_attr, u, batch_s,
           W1a, b1a, W1b, b1b, W2a, b2a, W2b, b2b):
    src = edge_index[0]
    tgt = edge_index[1]

    xt_pad = jnp.pad(x_t, ((0, 0), (0, 3)))
    u_pad = jnp.pad(u, ((0, 0), (0, 6)))
    w1aT_p = jnp.pad(W1a.T, ((0, 1), (0, 1)))
    w1bT_p = jnp.pad(W1b.T, ((0, 1), (0, 1)))
    b1aT_p = jnp.pad(b1a, (0, 1)).reshape(16, 1)
    b1bT_p = jnp.pad(b1b, (0, 1)).reshape(16, 1)
    w2aT_p = jnp.pad(W2a.T, ((0, 6), (0, 7)))
    b2aT_p = jnp.pad(b2a, (0, 6)).reshape(16, 1)
    w2bT_p = jnp.pad(W2b.T, ((0, 6), (0, 6)))
    b2bT_p = jnp.pad(b2b, (0, 6)).reshape(16, 1)

    xt_g, u_g, bids, bsrc, bcnt = _run_stage_a(tgt, src, xt_pad, u_pad,
                                               batch_s)
    xt_gT = xt_g.T
    eaT = edge_attr.T
    out_p = _run_stage_b(xt_gT, eaT, w1aT_p, b1aT_p, w1bT_p, b1bT_p)
    s1, s2, s3, s4 = _run_stage_c(out_p, bids, bsrc, bcnt)
    zpad = ((0, 0), (0, NP - N_S))
    s1T, s2T, s3T, s4T = (
        jnp.pad(jnp.reshape(t[:, :QN, :], (N_S, 16)).T, zpad)
        for t in (s1, s2, s3, s4))
    xsT = jnp.pad(x_s.T, zpad)
    ugT = jnp.pad(u_g.T[:10], zpad)
    outT2 = _run_stage_d(s1T, s2T, s3T, s4T, xsT, ugT,
                         w2aT_p, b2aT_p, w2bT_p, b2bT_p)
    return outT2[:10, :N_S].T
